# baseline (speedup 1.0000x reference)
import functools

import jax
import jax.numpy as jnp
from jax import lax
from jax.experimental import pallas as pl
from jax.experimental.pallas import tpu as pltpu
from jax.experimental.pallas import tpu_sc as plsc

B = 4096
L = 50
D = 64
NW = 32
BPW = B // NW
V = 100000
QA = 25088
SPLIT = 2 * QA
_TC = 3584
_TCG = QA // _TC


def _zero_acc(acc):
    zero = jnp.zeros((16,), jnp.float32)

    def body(b, _):
        for d in range(4):
            acc[b, pl.ds(d * 16, 16)] = zero
        return 0

    lax.fori_loop(0, BPW, body, 0)


def _drain(w_hbm, ids_tv, acc, sem):
    def body(l, _):
        pltpu.make_async_copy(w_hbm.at[ids_tv.at[0]], acc, sem).wait()
        return 0

    lax.fori_loop(0, L, body, 0)


def _pool_a(ids_hbm, w_hbm, out_hbm, ids_tv, acc, sem):
    wid = lax.axis_index("s") * 2 + lax.axis_index("c")
    pltpu.sync_copy(ids_hbm.at[:, pl.ds(wid * BPW, BPW)], ids_tv)
    _zero_acc(acc)

    def fire_body(l, _):
        for g in range(BPW // 16):
            v = ids_tv[l, pl.ds(g * 16, 16)]
            va = jnp.where(v >= QA, v * 2 - (2 * QA - 1), v * 2)
            va = jnp.where(v >= SPLIT, 0, va)
            ids_tv[l, pl.ds(g * 16, 16)] = va
        pltpu.async_copy(w_hbm.at[ids_tv.at[l]], acc, sem, add=True)
        return 0

    lax.fori_loop(0, L, fire_body, 0)
    _drain(w_hbm, ids_tv, acc, sem)
    pltpu.sync_copy(acc, out_hbm.at[pl.ds(wid * BPW, BPW)])


def _pool_b(ids_hbm, w_hbm, part_hbm, out_hbm, ids_tv, acc, inv_v, sem):
    wid = lax.axis_index("s") * 2 + lax.axis_index("c")
    pltpu.sync_copy(ids_hbm.at[:, pl.ds(wid * BPW, BPW)], ids_tv)
    pltpu.sync_copy(part_hbm.at[pl.ds(wid * BPW, BPW)], acc)

    zero = jnp.zeros((16,), jnp.float32)
    for g in range(BPW // 16):
        inv_v[pl.ds(g * 16, 16)] = zero

    def fire_body(l, _):
        for g in range(BPW // 16):
            v = ids_tv[l, pl.ds(g * 16, 16)]
            plsc.addupdate(
                inv_v.at[pl.ds(g * 16, 16)],
                jnp.where(v != 0, 1.0, 0.0).astype(jnp.float32),
            )
            u = v - SPLIT
            vb = jnp.where(u >= QA, u * 2 - (2 * QA - 1), u * 2)
            vb = jnp.where(u < 0, 2 * QA - 1, vb)
            ids_tv[l, pl.ds(g * 16, 16)] = vb
        pltpu.async_copy(w_hbm.at[ids_tv.at[l]], acc, sem, add=True)
        return 0

    lax.fori_loop(0, L, fire_body, 0)

    for g in range(BPW // 16):
        inv_v[pl.ds(g * 16, 16)] = 1.0 / (inv_v[pl.ds(g * 16, 16)] + 1e-8)

    _drain(w_hbm, ids_tv, acc, sem)

    def scale_body(b, _):
        iv = jnp.full((16,), inv_v[pl.ds(b, 16)][0])
        for d in range(4):
            acc[b, pl.ds(d * 16, 16)] = acc[b, pl.ds(d * 16, 16)] * iv
        return 0

    lax.fori_loop(0, BPW, scale_body, 0)
    pltpu.sync_copy(acc, out_hbm.at[pl.ds(wid * BPW, BPW)])


def _tc_half(wt, base):
    base_b = (base + _TCG) * _TC

    def body(xa_ref, xb_ref, y_ref):
        y_ref[:, 0:D] = xa_ref[...].T
        xbt = xb_ref[...].T
        if base_b + QA > V:
            j = pl.program_id(0)
            v = base_b + j * _TC + lax.broadcasted_iota(jnp.int32, (_TC, D), 0)
            xbt = jnp.where(v < V, xbt, 0.0)
        y_ref[:, D : 2 * D] = xbt

    return pl.pallas_call(
        body,
        grid=(_TCG,),
        in_specs=[
            pl.BlockSpec((D, _TC), lambda j, base=base: (0, j + base)),
            pl.BlockSpec((D, _TC), lambda j, base=base: (0, j + base + _TCG)),
        ],
        out_specs=pl.BlockSpec((_TC, 2 * D), lambda j: (j, 0)),
        out_shape=jax.ShapeDtypeStruct((QA, 2 * D), jnp.float32),
    )(wt, wt)


def kernel(ids, weight):
    ids_t = ids.astype(jnp.int32).T
    wt = weight.astype(jnp.float32).T
    w_a = _tc_half(wt, 0).reshape(2 * QA, D)
    w_b = _tc_half(wt, 2 * _TCG).reshape(2 * QA, D)

    mesh = plsc.VectorSubcoreMesh(core_axis_name="c", subcore_axis_name="s")
    params = pltpu.CompilerParams(
        needs_layout_passes=False, use_tc_tiling_on_sc=False
    )
    run_a = functools.partial(
        pl.kernel,
        mesh=mesh,
        compiler_params=params,
        out_type=jax.ShapeDtypeStruct((B, D), jnp.float32),
        scratch_types=[
            pltpu.VMEM((L, BPW), jnp.int32),
            pltpu.VMEM((BPW, D), jnp.float32),
            pltpu.SemaphoreType.DMA,
        ],
    )(_pool_a)
    partial_sums = run_a(ids_t, w_a)
    run_b = functools.partial(
        pl.kernel,
        mesh=mesh,
        compiler_params=params,
        out_type=jax.ShapeDtypeStruct((B, D), jnp.float32),
        scratch_types=[
            pltpu.VMEM((L, BPW), jnp.int32),
            pltpu.VMEM((BPW, D), jnp.float32),
            pltpu.VMEM((BPW + 16,), jnp.float32),
            pltpu.SemaphoreType.DMA,
        ],
    )(_pool_b)
    return run_b(ids_t, w_b, partial_sums)


# speedup vs baseline: 57.6300x; 57.6300x over previous
import functools

import jax
import jax.numpy as jnp
from jax import lax
from jax.experimental import pallas as pl
from jax.experimental.pallas import tpu as pltpu
from jax.experimental.pallas import tpu_sc as plsc

B = 4096
L = 50
D = 64
NW = 32
BPW = B // NW
HALF_V = 50176


def _pool_kernel(ids_hbm, w_hbm, out_hbm, ids_tv, acc, inv_v, sem):
    wid = lax.axis_index("s") * 2 + lax.axis_index("c")
    pltpu.sync_copy(ids_hbm.at[:, pl.ds(wid * BPW, BPW)], ids_tv)

    zero = jnp.zeros((16,), jnp.float32)

    def zero_body(b, _):
        for d in range(4):
            acc[b, pl.ds(d * 16, 16)] = zero
        return 0

    lax.fori_loop(0, BPW, zero_body, 0)

    def fire_body(l, _):
        for g in range(BPW // 16):
            v = ids_tv[l, pl.ds(g * 16, 16)]
            v2 = jnp.where(v >= HALF_V, v * 2 - (2 * HALF_V - 1), v * 2)
            ids_tv[l, pl.ds(g * 16, 16)] = v2
        pltpu.async_copy(w_hbm.at[ids_tv.at[l]], acc, sem, add=True)
        return 0

    lax.fori_loop(0, L, fire_body, 0)

    for g in range(BPW // 16):
        def cnt_body(l, cnt):
            v = ids_tv[l, pl.ds(g * 16, 16)]
            return cnt + jnp.where(v != 0, 1.0, 0.0).astype(jnp.float32)

        cnt = lax.fori_loop(0, L, cnt_body, jnp.zeros((16,), jnp.float32))
        inv_v[pl.ds(g * 16, 16)] = 1.0 / (cnt + 1e-8)

    def drain_body(l, _):
        pltpu.make_async_copy(w_hbm.at[ids_tv.at[0]], acc, sem).wait()
        return 0

    lax.fori_loop(0, L, drain_body, 0)

    def scale_body(b, _):
        iv = jnp.full((16,), inv_v[pl.ds(b, 16)][0])
        for d in range(4):
            acc[b, pl.ds(d * 16, 16)] = acc[b, pl.ds(d * 16, 16)] * iv
        return 0

    lax.fori_loop(0, BPW, scale_body, 0)
    pltpu.sync_copy(acc, out_hbm.at[pl.ds(wid * BPW, BPW)])


V = 100000
_TC = 7168
_TCG = HALF_V // _TC


def _tc_transpose_body(xa_ref, xb_ref, y_ref):
    y_ref[:, 0:D] = xa_ref[...].T
    y_ref[:, D : 2 * D] = xb_ref[...].T


def _tc_transpose(wt):
    return pl.pallas_call(
        _tc_transpose_body,
        grid=(_TCG,),
        in_specs=[
            pl.BlockSpec((D, _TC), lambda j: (0, j)),
            pl.BlockSpec((D, _TC), lambda j: (0, j + _TCG)),
        ],
        out_specs=pl.BlockSpec((_TC, 2 * D), lambda j: (j, 0)),
        out_shape=jax.ShapeDtypeStruct((HALF_V, 2 * D), jnp.float32),
    )(wt, wt)


def kernel(ids, weight):
    ids_t = ids.astype(jnp.int32).T
    weight = _tc_transpose(weight.astype(jnp.float32).T).reshape(2 * HALF_V, D)
    mesh = plsc.VectorSubcoreMesh(core_axis_name="c", subcore_axis_name="s")
    run = functools.partial(
        pl.kernel,
        mesh=mesh,
        compiler_params=pltpu.CompilerParams(
            needs_layout_passes=False, use_tc_tiling_on_sc=False
        ),
        out_type=jax.ShapeDtypeStruct((B, D), jnp.float32),
        scratch_types=[
            pltpu.VMEM((L, BPW), jnp.int32),
            pltpu.VMEM((BPW, D), jnp.float32),
            pltpu.VMEM((BPW + 16,), jnp.float32),
            pltpu.SemaphoreType.DMA,
        ],
    )(_pool_kernel)
    return run(ids_t, weight)
